# baseline (speedup 1.0000x reference)
import functools

import jax
import jax.numpy as jnp
from jax import lax
from jax.experimental import pallas as pl
from jax.experimental.pallas import tpu as pltpu
from jax.experimental.pallas import tpu_sc as plsc

_N = 10000
_E = 320000
_D = 128
_DH = 64
_NC = 2
_NS = 16
_EPS = _E // _NS
_CH = 125
_NCH = _EPS // _CH
_NP = 10240
_RPS = _NP // _NS
_CW = 16


def _sc_aggregate(x0, x1, src2d, dst2d, with_cnt):
    mesh = plsc.VectorSubcoreMesh(core_axis_name="c", subcore_axis_name="s")

    out_type = [jax.ShapeDtypeStruct((_NC, _NP, _DH), jnp.float32)]
    scratch = [
        pltpu.VMEM((_NCH, _CH), jnp.int32),
        pltpu.VMEM((_NCH, _CH), jnp.int32),
        pltpu.VMEM((4, _CH, _DH), jnp.float32),
        pltpu.VMEM((_CH, _CW), jnp.float32),
        pltpu.VMEM_SHARED((_NP, _DH), jnp.float32),
        pltpu.SemaphoreType.DMA,
        pltpu.SemaphoreType.DMA,
        pltpu.SemaphoreType.DMA,
        pltpu.SemaphoreType.DMA,
        pltpu.SemaphoreType.DMA,
        pltpu.SemaphoreType.DMA,
        pltpu.SemaphoreType.DMA,
        pltpu.SemaphoreType.DMA,
        pltpu.SemaphoreType.DMA,
    ]
    if with_cnt:
        out_type.append(jax.ShapeDtypeStruct((_NC, _NP, _CW), jnp.float32))
        scratch.insert(5, pltpu.VMEM_SHARED((_NP, _CW), jnp.float32))

    @functools.partial(
        pl.kernel,
        out_type=tuple(out_type),
        mesh=mesh,
        scratch_types=tuple(scratch),
        compiler_params=pltpu.CompilerParams(use_tc_tiling_on_sc=False),
    )
    def agg_kernel(x0_hbm, x1_hbm, src_hbm, dst_hbm, z64_hbm, z16_hbm,
                   o16_hbm, agg_hbm, *rest):
        if with_cnt:
            cnt_hbm, sidx_v, didx_v, rows_v, ones_v, agg_sh, cnt_sh, *sems = rest
        else:
            sidx_v, didx_v, rows_v, ones_v, agg_sh, *sems = rest
        gsems, ssems, osem = sems[0:4], sems[4:8], sems[8]
        cid = lax.axis_index("c")
        sid = lax.axis_index("s")

        pltpu.sync_copy(src_hbm.at[pl.ds(sid * _NCH, _NCH)], sidx_v)
        pltpu.sync_copy(dst_hbm.at[pl.ds(sid * _NCH, _NCH)], didx_v)

        pltpu.sync_copy(z64_hbm, agg_sh.at[pl.ds(sid * _RPS, _RPS)])
        if with_cnt:
            pltpu.sync_copy(z16_hbm, cnt_sh.at[pl.ds(sid * _RPS, _RPS)])
            pltpu.sync_copy(o16_hbm, ones_v)

        def gather_start(buf, j):
            @pl.when(cid == 0)
            def _():
                pltpu.async_copy(x0_hbm.at[sidx_v.at[j]], rows_v.at[buf],
                                 gsems[buf])

            @pl.when(cid == 1)
            def _():
                pltpu.async_copy(x1_hbm.at[sidx_v.at[j]], rows_v.at[buf],
                                 gsems[buf])

        def gather_wait(buf):
            pltpu.make_async_copy(
                x0_hbm.at[pl.ds(0, _CH)], rows_v.at[buf], gsems[buf]).wait()

        def scatter_wait(buf):
            pltpu.make_async_copy(
                x0_hbm.at[pl.ds(0, _CH)], rows_v.at[buf], ssems[buf]).wait()

        def ones_wait():
            pltpu.make_async_copy(
                x0_hbm.at[pl.ds(0, _CH), pl.ds(0, _CW)], ones_v, osem).wait()

        gather_start(0, 0)
        gather_start(1, 1)
        plsc.subcore_barrier()

        def edge_quad(i, _):
            for b in range(4):
                j = 4 * i + b
                nb = (b + 2) % 4
                gather_wait(b)
                pltpu.async_copy(rows_v.at[b], agg_sh.at[didx_v.at[j]],
                                 ssems[b], add=True)
                if with_cnt:
                    @pl.when((cid == b % 2) & (i > 0))
                    def _():
                        ones_wait()

                    @pl.when(cid == b % 2)
                    def _():
                        pltpu.async_copy(ones_v, cnt_sh.at[didx_v.at[j]],
                                         osem, add=True)
                if b < 2:
                    @pl.when(i > 0)
                    def _():
                        scatter_wait(nb)
                else:
                    scatter_wait(nb)
                gather_start(nb, jnp.where(j + 2 < _NCH, j + 2, b))
            return 0
        lax.fori_loop(0, _NCH // 4, edge_quad, 0)

        gather_wait(0)
        gather_wait(1)
        scatter_wait(2)
        scatter_wait(3)
        if with_cnt:
            ones_wait()
            ones_wait()

        plsc.subcore_barrier()

        off = sid * _RPS
        pltpu.sync_copy(agg_sh.at[pl.ds(off, _RPS)],
                        agg_hbm.at[cid, pl.ds(off, _RPS)])
        if with_cnt:
            pltpu.sync_copy(cnt_sh.at[pl.ds(off, _RPS)],
                            cnt_hbm.at[cid, pl.ds(off, _RPS)])

    z64 = jnp.zeros((_RPS, _DH), jnp.float32)
    z16 = jnp.zeros((_RPS, _CW), jnp.float32)
    o16 = jnp.ones((_CH, _CW), jnp.float32)
    out = agg_kernel(x0, x1, src2d, dst2d, z64, z16, o16)
    return out if with_cnt else out[0]


def _dense_xr(xcat, Wr, b):
    bn = 1000
    dims = (((1,), (1,)), ((), ()))

    def body(x_ref, wr_ref, b_ref, o_ref):
        acc = b_ref[...]
        for c in range(_NC):
            cols = pl.ds(c * _DH, _DH)
            acc = acc + lax.dot_general(x_ref[c], wr_ref[:, cols],
                                        dims, preferred_element_type=jnp.float32)
        o_ref[...] = acc

    return pl.pallas_call(
        body,
        grid=(_N // bn,),
        in_specs=[
            pl.BlockSpec((_NC, bn, _DH), lambda i: (0, i, 0)),
            pl.BlockSpec((_D, _D), lambda i: (0, 0)),
            pl.BlockSpec((1, _D), lambda i: (0, 0)),
        ],
        out_specs=pl.BlockSpec((bn, _D), lambda i: (i, 0)),
        out_shape=jax.ShapeDtypeStruct((_N, _D), jnp.float32),
    )(xcat, Wr, b)


def _dense_combine(agg, cnt, xr, Wl, relu, split_out):
    bn = 1000
    dims = (((1,), (1,)), ((), ()))

    def body(agg_ref, cnt_ref, xr_ref, wl_ref, o_ref):
        cnt = cnt_ref[0, :, 0:1] + cnt_ref[1, :, 0:1]
        inv = 1.0 / jnp.maximum(cnt, 1.0)
        acc = xr_ref[...]
        for c in range(_NC):
            cols = pl.ds(c * _DH, _DH)
            acc = acc + lax.dot_general(agg_ref[c] * inv, wl_ref[:, cols],
                                        dims, preferred_element_type=jnp.float32)
        if relu:
            acc = jnp.maximum(acc, 0.0)
        if split_out:
            o_ref[0] = acc[:, : _DH]
            o_ref[1] = acc[:, _DH:]
        else:
            o_ref[...] = acc

    if split_out:
        out_shape = jax.ShapeDtypeStruct((_NC, _N, _DH), jnp.float32)
        out_specs = pl.BlockSpec((_NC, bn, _DH), lambda i: (0, i, 0))
    else:
        out_shape = jax.ShapeDtypeStruct((_N, _D), jnp.float32)
        out_specs = pl.BlockSpec((bn, _D), lambda i: (i, 0))

    return pl.pallas_call(
        body,
        grid=(_N // bn,),
        in_specs=[
            pl.BlockSpec((_NC, bn, _DH), lambda i: (0, i, 0)),
            pl.BlockSpec((_NC, bn, _CW), lambda i: (0, i, 0)),
            pl.BlockSpec((bn, _D), lambda i: (i, 0)),
            pl.BlockSpec((_D, _D), lambda i: (0, 0)),
        ],
        out_specs=out_specs,
        out_shape=out_shape,
    )(agg, cnt, xr, Wl)


def kernel(x, edge_index, W1l, b1l, W1r, W2l, b2l, W2r):
    x = x.astype(jnp.float32)
    src = edge_index[0].astype(jnp.int32).reshape(_E // _CH, _CH)
    dst = edge_index[1].astype(jnp.int32).reshape(_E // _CH, _CH)
    xcat = jnp.transpose(x.reshape(_N, _NC, _DH), (1, 0, 2))
    agg1, cnt = _sc_aggregate(xcat[0], xcat[1], src, dst, with_cnt=True)
    xr1 = _dense_xr(xcat, W1r, b1l.reshape(1, _D))
    hcat = _dense_combine(agg1, cnt, xr1, W1l, relu=True, split_out=True)
    agg2 = _sc_aggregate(hcat[0], hcat[1], src, dst, with_cnt=False)
    xr2 = _dense_xr(hcat, W2r, b2l.reshape(1, _D))
    out = _dense_combine(agg2, cnt, xr2, W2l, relu=False, split_out=False)
    return out


# speedup vs baseline: 1.0279x; 1.0279x over previous
import functools

import jax
import jax.numpy as jnp
from jax import lax
from jax.experimental import pallas as pl
from jax.experimental.pallas import tpu as pltpu
from jax.experimental.pallas import tpu_sc as plsc

_N = 10000
_E = 320000
_D = 128
_DH = 64
_NC = 2
_NS = 16
_EPS = _E // _NS
_CH = 125
_NCH = _EPS // _CH
_NP = 10240
_RPS = _NP // _NS
_CW = 16


def _sc_aggregate(x0, x1, src2d, dst2d, with_cnt):
    mesh = plsc.VectorSubcoreMesh(core_axis_name="c", subcore_axis_name="s")

    out_type = [jax.ShapeDtypeStruct((_NC, _NP, _DH), jnp.float32)]
    scratch = [
        pltpu.VMEM((_NCH, _CH), jnp.int32),
        pltpu.VMEM((_NCH, _CH), jnp.int32),
        pltpu.VMEM((4, _CH, _DH), jnp.float32),
        pltpu.VMEM((_CH, _CW), jnp.float32),
        pltpu.VMEM_SHARED((_NP, _DH), jnp.float32),
        pltpu.SemaphoreType.DMA,
        pltpu.SemaphoreType.DMA,
        pltpu.SemaphoreType.DMA,
        pltpu.SemaphoreType.DMA,
        pltpu.SemaphoreType.DMA,
        pltpu.SemaphoreType.DMA,
        pltpu.SemaphoreType.DMA,
        pltpu.SemaphoreType.DMA,
        pltpu.SemaphoreType.DMA,
    ]
    if with_cnt:
        out_type.append(jax.ShapeDtypeStruct((_NC, _NP, _CW), jnp.float32))
        scratch.insert(5, pltpu.VMEM_SHARED((_NP, _CW), jnp.float32))

    @functools.partial(
        pl.kernel,
        out_type=tuple(out_type),
        mesh=mesh,
        scratch_types=tuple(scratch),
        compiler_params=pltpu.CompilerParams(use_tc_tiling_on_sc=False),
    )
    def agg_kernel(x0_hbm, x1_hbm, src_hbm, dst_hbm, agg_hbm, *rest):
        if with_cnt:
            cnt_hbm, sidx_v, didx_v, rows_v, ones_v, agg_sh, cnt_sh, *sems = rest
        else:
            sidx_v, didx_v, rows_v, ones_v, agg_sh, *sems = rest
        gsems, ssems, osem = sems[0:4], sems[4:8], sems[8]
        cid = lax.axis_index("c")
        sid = lax.axis_index("s")

        pltpu.sync_copy(src_hbm.at[pl.ds(sid * _NCH, _NCH)], sidx_v)
        pltpu.sync_copy(dst_hbm.at[pl.ds(sid * _NCH, _NCH)], didx_v)

        def zrow(r, _):
            for c in range(_DH // 16):
                rows_v[0, r, pl.ds(c * 16, 16)] = jnp.zeros((16,), jnp.float32)
            return 0
        lax.fori_loop(0, _CH, zrow, 0)

        def zcnt(r, _):
            ones_v[r, :] = jnp.zeros((16,), jnp.float32)
            return 0
        lax.fori_loop(0, _CH, zcnt, 0)

        for k, w in ((0, 120), (1, 120), (2, 120), (3, 120), (4, 120), (5, 40)):
            off = sid * _RPS + k * 120
            pltpu.sync_copy(rows_v.at[0, pl.ds(0, w)], agg_sh.at[pl.ds(off, w)])
            if with_cnt:
                pltpu.sync_copy(ones_v.at[pl.ds(0, w)], cnt_sh.at[pl.ds(off, w)])

        def ocnt(r, _):
            ones_v[r, :] = jnp.ones((16,), jnp.float32)
            return 0
        lax.fori_loop(0, _CH, ocnt, 0)

        def gather_start(buf, j):
            @pl.when(cid == 0)
            def _():
                pltpu.async_copy(x0_hbm.at[sidx_v.at[j]], rows_v.at[buf],
                                 gsems[buf])

            @pl.when(cid == 1)
            def _():
                pltpu.async_copy(x1_hbm.at[sidx_v.at[j]], rows_v.at[buf],
                                 gsems[buf])

        def gather_wait(buf):
            pltpu.make_async_copy(
                x0_hbm.at[pl.ds(0, _CH)], rows_v.at[buf], gsems[buf]).wait()

        def scatter_wait(buf):
            pltpu.make_async_copy(
                x0_hbm.at[pl.ds(0, _CH)], rows_v.at[buf], ssems[buf]).wait()

        def ones_wait():
            pltpu.make_async_copy(
                x0_hbm.at[pl.ds(0, _CH), pl.ds(0, _CW)], ones_v, osem).wait()

        gather_start(0, 0)
        gather_start(1, 1)
        plsc.subcore_barrier()

        def edge_quad(i, _):
            for b in range(4):
                j = 4 * i + b
                nb = (b + 2) % 4
                gather_wait(b)
                pltpu.async_copy(rows_v.at[b], agg_sh.at[didx_v.at[j]],
                                 ssems[b], add=True)
                if with_cnt:
                    @pl.when((cid == b % 2) & (i > 0))
                    def _():
                        ones_wait()

                    @pl.when(cid == b % 2)
                    def _():
                        pltpu.async_copy(ones_v, cnt_sh.at[didx_v.at[j]],
                                         osem, add=True)
                if b < 2:
                    @pl.when(i > 0)
                    def _():
                        scatter_wait(nb)
                else:
                    scatter_wait(nb)
                gather_start(nb, jnp.where(j + 2 < _NCH, j + 2, b))
            return 0
        lax.fori_loop(0, _NCH // 4, edge_quad, 0)

        gather_wait(0)
        gather_wait(1)
        scatter_wait(2)
        scatter_wait(3)
        if with_cnt:
            ones_wait()
            ones_wait()

        plsc.subcore_barrier()

        off = sid * _RPS
        pltpu.sync_copy(agg_sh.at[pl.ds(off, _RPS)],
                        agg_hbm.at[cid, pl.ds(off, _RPS)])
        if with_cnt:
            pltpu.sync_copy(cnt_sh.at[pl.ds(off, _RPS)],
                            cnt_hbm.at[cid, pl.ds(off, _RPS)])

    out = agg_kernel(x0, x1, src2d, dst2d)
    return out if with_cnt else out[0]


def _dense_xr(xcat, Wr, b):
    bn = 1000
    dims = (((1,), (1,)), ((), ()))

    def body(x_ref, wr_ref, b_ref, o_ref):
        acc = b_ref[...]
        for c in range(_NC):
            cols = pl.ds(c * _DH, _DH)
            acc = acc + lax.dot_general(x_ref[c], wr_ref[:, cols],
                                        dims, preferred_element_type=jnp.float32)
        o_ref[...] = acc

    return pl.pallas_call(
        body,
        grid=(_N // bn,),
        in_specs=[
            pl.BlockSpec((_NC, bn, _DH), lambda i: (0, i, 0)),
            pl.BlockSpec((_D, _D), lambda i: (0, 0)),
            pl.BlockSpec((1, _D), lambda i: (0, 0)),
        ],
        out_specs=pl.BlockSpec((bn, _D), lambda i: (i, 0)),
        out_shape=jax.ShapeDtypeStruct((_N, _D), jnp.float32),
    )(xcat, Wr, b)


def _dense_combine(agg, cnt, xr, Wl, relu, split_out):
    bn = 1000
    dims = (((1,), (1,)), ((), ()))

    def body(agg_ref, cnt_ref, xr_ref, wl_ref, o_ref):
        cnt = cnt_ref[0, :, 0:1] + cnt_ref[1, :, 0:1]
        inv = 1.0 / jnp.maximum(cnt, 1.0)
        acc = xr_ref[...]
        for c in range(_NC):
            cols = pl.ds(c * _DH, _DH)
            acc = acc + lax.dot_general(agg_ref[c] * inv, wl_ref[:, cols],
                                        dims, preferred_element_type=jnp.float32)
        if relu:
            acc = jnp.maximum(acc, 0.0)
        if split_out:
            o_ref[0] = acc[:, : _DH]
            o_ref[1] = acc[:, _DH:]
        else:
            o_ref[...] = acc

    if split_out:
        out_shape = jax.ShapeDtypeStruct((_NC, _N, _DH), jnp.float32)
        out_specs = pl.BlockSpec((_NC, bn, _DH), lambda i: (0, i, 0))
    else:
        out_shape = jax.ShapeDtypeStruct((_N, _D), jnp.float32)
        out_specs = pl.BlockSpec((bn, _D), lambda i: (i, 0))

    return pl.pallas_call(
        body,
        grid=(_N // bn,),
        in_specs=[
            pl.BlockSpec((_NC, bn, _DH), lambda i: (0, i, 0)),
            pl.BlockSpec((_NC, bn, _CW), lambda i: (0, i, 0)),
            pl.BlockSpec((bn, _D), lambda i: (i, 0)),
            pl.BlockSpec((_D, _D), lambda i: (0, 0)),
        ],
        out_specs=out_specs,
        out_shape=out_shape,
    )(agg, cnt, xr, Wl)


def kernel(x, edge_index, W1l, b1l, W1r, W2l, b2l, W2r):
    x = x.astype(jnp.float32)
    src = edge_index[0].astype(jnp.int32).reshape(_E // _CH, _CH)
    dst = edge_index[1].astype(jnp.int32).reshape(_E // _CH, _CH)
    xcat = jnp.transpose(x.reshape(_N, _NC, _DH), (1, 0, 2))
    agg1, cnt = _sc_aggregate(xcat[0], xcat[1], src, dst, with_cnt=True)
    xr1 = _dense_xr(xcat, W1r, b1l.reshape(1, _D))
    hcat = _dense_combine(agg1, cnt, xr1, W1l, relu=True, split_out=True)
    agg2 = _sc_aggregate(hcat[0], hcat[1], src, dst, with_cnt=False)
    xr2 = _dense_xr(hcat, W2r, b2l.reshape(1, _D))
    out = _dense_combine(agg2, cnt, xr2, W2l, relu=False, split_out=False)
    return out
